# baseline (device time: 21378 ns/iter reference)
import jax
import jax.numpy as jnp
from jax import lax
from jax.experimental import pallas as pl
from jax.experimental.pallas import tpu as pltpu

N_Z = 4
K = 16
LANES = 128


def _topk_rows(x, k):
    m, n = x.shape
    iota = lax.broadcasted_iota(jnp.int32, (m, n), 1).astype(jnp.float32)
    x = x + iota * jnp.float32(1e-6)
    neg = jnp.float32(jnp.finfo(jnp.float32).min)
    cols = []
    for _ in range(k):
        mx = jnp.max(x, axis=1, keepdims=True)
        cols.append(mx)
        x = jnp.where(x == mx, neg, x)
    return jnp.concatenate(cols, axis=1)


def _extract_from_stacks(r1, r2, r3, k):
    m = r1.shape[0]
    lane = lax.broadcasted_iota(jnp.int32, (m, LANES), 1).astype(jnp.float32)
    eps = lane * jnp.float32(1e-6)
    r1 = r1.astype(jnp.float32) + eps
    r2 = r2.astype(jnp.float32) + eps
    r3 = r3.astype(jnp.float32) + eps
    negf = jnp.float32(jnp.finfo(jnp.float32).min)
    cols = []
    for _ in range(k):
        mx = jnp.max(r1, axis=1, keepdims=True)
        cols.append(mx)
        sel = r1 == mx
        r1 = jnp.where(sel, r2, r1)
        r2 = jnp.where(sel, r3, r2)
        r3 = jnp.where(sel, negf, r3)
    return jnp.concatenate(cols, axis=1)


def kernel(x):
    m, n = x.shape
    n_chunks = n // LANES

    def body(x_blk, o_ref, r1_ref, r2_ref, r3_ref, cand_ref, send_sems, recv_sems):
        i = pl.program_id(0)
        negb = jnp.bfloat16(jnp.finfo(jnp.bfloat16).min)

        @pl.when(i == 0)
        def _():
            r1_ref[:, :] = jnp.full((m, LANES), negb, jnp.bfloat16)
            r2_ref[:, :] = jnp.full((m, LANES), negb, jnp.bfloat16)
            r3_ref[:, :] = jnp.full((m, LANES), negb, jnp.bfloat16)

        v = x_blk[:, :].astype(jnp.bfloat16)
        r1 = r1_ref[:, :]
        r2 = r2_ref[:, :]
        r3 = r3_ref[:, :]
        m1 = jnp.maximum(r1, v)
        s = jnp.minimum(r1, v)
        m2 = jnp.maximum(r2, s)
        s = jnp.minimum(r2, s)
        m3 = jnp.maximum(r3, s)
        r1_ref[:, :] = m1
        r2_ref[:, :] = m2
        r3_ref[:, :] = m3

        @pl.when(i == n_chunks - 1)
        def _():
            my_x = lax.axis_index("x")
            my_y = lax.axis_index("y")
            my_z = lax.axis_index("z")

            bsem = pltpu.get_barrier_semaphore()
            for dz in range(1, N_Z):
                pl.semaphore_signal(
                    bsem,
                    inc=1,
                    device_id=(my_x, my_y, (my_z + dz) % N_Z),
                    device_id_type=pl.DeviceIdType.MESH,
                )
            pl.semaphore_wait(bsem, N_Z - 1)

            cand_ref[0] = _extract_from_stacks(
                r1_ref[:, :], r2_ref[:, :], r3_ref[:, :], K
            ).astype(jnp.bfloat16)

            rdmas = []
            for dz in range(1, N_Z):
                rdma = pltpu.make_async_remote_copy(
                    src_ref=cand_ref.at[0],
                    dst_ref=cand_ref.at[N_Z - dz],
                    send_sem=send_sems.at[dz - 1],
                    recv_sem=recv_sems.at[N_Z - dz - 1],
                    device_id=(my_x, my_y, (my_z + dz) % N_Z),
                    device_id_type=pl.DeviceIdType.MESH,
                )
                rdma.start()
                rdmas.append(rdma)
            for rdma in rdmas:
                rdma.wait()

            merged = jnp.concatenate(
                [cand_ref[j] for j in range(N_Z)], axis=1
            )
            o_ref[:, :] = _topk_rows(merged.astype(jnp.float32), K)

    return pl.pallas_call(
        body,
        grid=(n_chunks,),
        out_shape=jax.ShapeDtypeStruct((m, K), jnp.float32),
        in_specs=[pl.BlockSpec((m, LANES), lambda i: (0, i))],
        out_specs=pl.BlockSpec((m, K), lambda i: (0, 0)),
        scratch_shapes=[
            pltpu.VMEM((m, LANES), jnp.bfloat16),
            pltpu.VMEM((m, LANES), jnp.bfloat16),
            pltpu.VMEM((m, LANES), jnp.bfloat16),
            pltpu.VMEM((N_Z, m, K), jnp.bfloat16),
            pltpu.SemaphoreType.DMA((N_Z - 1,)),
            pltpu.SemaphoreType.DMA((N_Z - 1,)),
        ],
        compiler_params=pltpu.CompilerParams(
            collective_id=0,
            dimension_semantics=("arbitrary",),
        ),
    )(x)


# device time: 17597 ns/iter; 1.2149x vs baseline; 1.2149x over previous
import jax
import jax.numpy as jnp
from jax import lax
from jax.experimental import pallas as pl
from jax.experimental.pallas import tpu as pltpu

N_Z = 4
K = 16
LANES = 128


def _topk_rows(x, k):
    m, n = x.shape
    iota = lax.broadcasted_iota(jnp.int32, (m, n), 1).astype(jnp.float32)
    x = x + iota * jnp.float32(1e-6)
    neg = jnp.float32(jnp.finfo(jnp.float32).min)
    cols = []
    for _ in range(k):
        mx = jnp.max(x, axis=1, keepdims=True)
        cols.append(mx)
        x = jnp.where(x == mx, neg, x)
    return jnp.concatenate(cols, axis=1)


def _local_candidates(x, k):
    m, n = x.shape
    neg = jnp.array(jnp.finfo(x.dtype).min, x.dtype)
    r1 = jnp.full((m, LANES), neg, x.dtype)
    r2 = r1
    r3 = r1
    for c in range(n // LANES):
        v = x[:, c * LANES : (c + 1) * LANES]
        m1 = jnp.maximum(r1, v)
        s = jnp.minimum(r1, v)
        m2 = jnp.maximum(r2, s)
        s = jnp.minimum(r2, s)
        m3 = jnp.maximum(r3, s)
        r1, r2, r3 = m1, m2, m3
    lane = lax.broadcasted_iota(jnp.int32, (m, LANES), 1).astype(jnp.float32)
    eps = lane * jnp.float32(1e-6)
    r1 = r1.astype(jnp.float32) + eps
    r2 = r2.astype(jnp.float32) + eps
    r3 = r3.astype(jnp.float32) + eps
    negf = jnp.float32(jnp.finfo(jnp.float32).min)
    cols = []
    for _ in range(k):
        mx = jnp.max(r1, axis=1, keepdims=True)
        cols.append(mx)
        sel = r1 == mx
        r1 = jnp.where(sel, r2, r1)
        r2 = jnp.where(sel, r3, r2)
        r3 = jnp.where(sel, negf, r3)
    return jnp.concatenate(cols, axis=1).astype(x.dtype)


def kernel(x):
    m, _ = x.shape

    def body(x_ref, o_ref, cand_ref, send_sems, recv_sems):
        my_x = lax.axis_index("x")
        my_y = lax.axis_index("y")
        my_z = lax.axis_index("z")

        bsem = pltpu.get_barrier_semaphore()
        for dz in range(1, N_Z):
            pl.semaphore_signal(
                bsem,
                inc=1,
                device_id=(my_x, my_y, (my_z + dz) % N_Z),
                device_id_type=pl.DeviceIdType.MESH,
            )
        pl.semaphore_wait(bsem, N_Z - 1)

        cand_ref[0] = _local_candidates(x_ref[:, :].astype(jnp.bfloat16), K)

        rdmas = []
        for dz in range(1, N_Z):
            rdma = pltpu.make_async_remote_copy(
                src_ref=cand_ref.at[0],
                dst_ref=cand_ref.at[N_Z - dz],
                send_sem=send_sems.at[dz - 1],
                recv_sem=recv_sems.at[N_Z - dz - 1],
                device_id=(my_x, my_y, (my_z + dz) % N_Z),
                device_id_type=pl.DeviceIdType.MESH,
            )
            rdma.start()
            rdmas.append(rdma)
        for rdma in rdmas:
            rdma.wait()

        merged = jnp.concatenate([cand_ref[j] for j in range(N_Z)], axis=1)
        o_ref[:, :] = _topk_rows(merged.astype(jnp.float32), K)

    return pl.pallas_call(
        body,
        out_shape=jax.ShapeDtypeStruct((m, K), jnp.float32),
        in_specs=[pl.BlockSpec(memory_space=pltpu.VMEM)],
        out_specs=pl.BlockSpec(memory_space=pltpu.VMEM),
        scratch_shapes=[
            pltpu.VMEM((N_Z, m, K), jnp.bfloat16),
            pltpu.SemaphoreType.DMA((N_Z - 1,)),
            pltpu.SemaphoreType.DMA((N_Z - 1,)),
        ],
        compiler_params=pltpu.CompilerParams(collective_id=0),
    )(x)
